# Initial kernel scaffold; baseline (speedup 1.0000x reference)
import jax
import jax.numpy as jnp
from jax.experimental import pallas as pl


def kernel(z2, params):
    raise NotImplementedError("write your pallas kernel here")


# speedup vs baseline: 1.8019x; 1.8019x over previous
import math

import jax
import jax.numpy as jnp
from jax.experimental import pallas as pl
from jax.experimental.pallas import tpu as pltpu

_F32 = jnp.float32
_BF16 = jnp.bfloat16
_NEG = -1e30
_VMEM_LIMIT = 56 * 1024 * 1024


def _sigmoid(x):
    return 1.0 / (1.0 + jnp.exp(-x))


def _tanh(x):
    return 1.0 - 2.0 / (jnp.exp(2.0 * x) + 1.0)


def _erf(x):
    a1, a2, a3 = 0.254829592, -0.284496736, 1.421413741
    a4, a5, p = -1.453152027, 1.061405429, 0.3275911
    ax = jnp.abs(x)
    t = 1.0 / (1.0 + p * ax)
    poly = ((((a5 * t + a4) * t + a3) * t + a2) * t + a1) * t
    y = 1.0 - poly * jnp.exp(-ax * ax)
    return jnp.where(x < 0.0, -y, y)


def _gelu(x):
    return 0.5 * x * (1.0 + _erf(x * (1.0 / math.sqrt(2.0))))


def _ln(x, g, b):
    mu = jnp.mean(x, axis=-1, keepdims=True)
    xc = x - mu
    var = jnp.mean(xc * xc, axis=-1, keepdims=True)
    return xc * jax.lax.rsqrt(var + 1e-5) * g + b


def _dot(a, b):
    return jnp.dot(a, b, preferred_element_type=_F32)


def _make_prep_body(nmem):
    def _prep_body(zt_ref, mem_ref, mkw_ref, mkb_ref, mvw_ref, mvb_ref,
                   inw_ref, inb_ref, mqw_ref, mqb_ref, w0a_ref, w0b_ref,
                   b0_ref, out_ref, keys_scr, vals_scr):
        tb, bsz, d2 = zt_ref.shape
        dm = mem_ref.shape[1]

        @pl.when(pl.program_id(1) == 0)
        def _():
            memb = mem_ref[...].astype(_BF16)
            keys_scr[...] = (_dot(memb, mkw_ref[...])
                             + mkb_ref[...]).astype(_BF16)
            vals_scr[...] = (_dot(memb, mvw_ref[...])
                             + mvb_ref[...]).astype(_BF16)

        x = zt_ref[...].reshape(tb * bsz, d2).astype(_BF16)
        z3 = _dot(x, inw_ref[...]) + inb_ref[...]
        z3b = z3.astype(_BF16)
        q = (_dot(z3b, mqw_ref[...]) + mqb_ref[...]).astype(_BF16)
        s = jax.lax.dot_general(q, keys_scr[...], (((1,), (1,)), ((), ())),
                                preferred_element_type=_F32)
        s = s * (1.0 / math.sqrt(dm))
        lane = jax.lax.broadcasted_iota(jnp.int32, s.shape, 1)
        s = jnp.where(lane < nmem, s, _NEG)
        m = jnp.max(s, axis=-1, keepdims=True)
        e = jnp.exp(s - m)
        attn = (e / jnp.sum(e, axis=-1, keepdims=True)).astype(_BF16)
        read = _dot(attn, vals_scr[...]).astype(_BF16)
        xg = _dot(z3b, w0a_ref[...]) + _dot(read, w0b_ref[...]) + b0_ref[...]
        out_ref[...] = xg.reshape(tb, bsz, out_ref.shape[2])
    return _prep_body


def _scan_body(xg_ref, w0_ref, w1_ref, b1_ref, out_ref,
               h0_scr, c0_scr, h1_scr, c1_scr):
    h = out_ref.shape[2]

    @pl.when(pl.program_id(1) == 0)
    def _():
        h0_scr[...] = jnp.zeros_like(h0_scr)
        c0_scr[...] = jnp.zeros_like(c0_scr)
        h1_scr[...] = jnp.zeros_like(h1_scr)
        c1_scr[...] = jnp.zeros_like(c1_scr)

    g0 = xg_ref[0] + _dot(h0_scr[...], w0_ref[...])
    i0 = _sigmoid(g0[:, :h])
    f0 = _sigmoid(g0[:, h:2 * h])
    gg0 = _tanh(g0[:, 2 * h:3 * h])
    o0 = _sigmoid(g0[:, 3 * h:])
    c0 = f0 * c0_scr[...] + i0 * gg0
    h0b = (o0 * _tanh(c0)).astype(_BF16)
    c0_scr[...] = c0
    h0_scr[...] = h0b

    hcat = jnp.concatenate([h0b, h1_scr[...]], axis=1)
    g1 = _dot(hcat, w1_ref[...]) + b1_ref[...]
    i1 = _sigmoid(g1[:, :h])
    f1 = _sigmoid(g1[:, h:2 * h])
    gg1 = _tanh(g1[:, 2 * h:3 * h])
    o1 = _sigmoid(g1[:, 3 * h:])
    c1 = f1 * c1_scr[...] + i1 * gg1
    h1 = o1 * _tanh(c1)
    c1_scr[...] = c1
    h1_scr[...] = h1.astype(_BF16)
    out_ref[0] = h1


def _enc_body(x_ref, e1w_ref, e1b_ref, g1_ref, bb1_ref, e2w_ref, e2b_ref,
              g2_ref, bb2_ref, e3w_ref, e3b_ref, hw_ref, hb_ref,
              enc_ref, head_ref):
    x = x_ref[...].astype(_BF16)
    h = _gelu(_ln(_dot(x, e1w_ref[...]) + e1b_ref[...],
                  g1_ref[...], bb1_ref[...]))
    h = _gelu(_ln(_dot(h.astype(_BF16), e2w_ref[...]) + e2b_ref[...],
                  g2_ref[...], bb2_ref[...]))
    enc = _dot(h.astype(_BF16), e3w_ref[...]) + e3b_ref[...]
    enc_ref[...] = enc
    hd = _dot(enc.astype(_BF16), hw_ref[...]) + hb_ref[...]
    lane = jax.lax.broadcasted_iota(jnp.int32, hd.shape, 1)
    head_ref[...] = jnp.where(lane < 3, hd, _sigmoid(hd))


def _pred_body(zf_ref, p1w_ref, p1b_ref, p2w_ref, p2b_ref, out_ref):
    h = _gelu(_dot(zf_ref[...].astype(_BF16), p1w_ref[...]) + p1b_ref[...])
    out_ref[...] = _dot(h.astype(_BF16), p2w_ref[...]) + p2b_ref[...]


def _vspec():
    return pl.BlockSpec(memory_space=pltpu.VMEM)


def kernel(z2, params):
    p = params
    bsz, t, d2 = z2.shape
    d3 = p['in_w'].shape[0]
    dm = p['mq_w'].shape[0]
    h4 = p['wih0'].shape[0]
    h = h4 // 4
    de1 = p['e1_w'].shape[0]
    de2 = p['e2_w'].shape[0]

    def bf(x):
        return x.astype(_BF16)

    def row(x):
        return x.reshape(1, -1)

    z2t = jnp.swapaxes(z2, 0, 1)
    mem = p['memory'][0]
    mem_pad = jnp.pad(mem, ((0, 128 - mem.shape[0]), (0, 0)))
    w0t = p['wih0'].T
    w1cat = jnp.concatenate([p['wih1'].T, p['whh1'].T], axis=0)
    headw = jnp.concatenate(
        [p['mood_w'], p['pers_w'],
         jnp.zeros((128 - p['mood_w'].shape[0] - p['pers_w'].shape[0], d3),
                   _F32)], axis=0)
    headb = jnp.concatenate(
        [p['mood_b'], p['pers_b'],
         jnp.zeros((128 - p['mood_b'].shape[0] - p['pers_b'].shape[0],),
                   _F32)], axis=0)

    tb = 32
    n1 = t // tb // 2
    xg0 = pl.pallas_call(
        _make_prep_body(mem.shape[0]),
        grid=(2, n1),
        in_specs=[pl.BlockSpec((tb, bsz, d2), lambda c, j: (c * n1 + j, 0, 0))]
        + [_vspec()] * 12,
        out_specs=pl.BlockSpec((tb, bsz, h4), lambda c, j: (c * n1 + j, 0, 0)),
        out_shape=jax.ShapeDtypeStruct((t, bsz, h4), _F32),
        scratch_shapes=[pltpu.VMEM((128, dm), _BF16),
                        pltpu.VMEM((128, dm), _BF16)],
        compiler_params=pltpu.CompilerParams(
            dimension_semantics=("parallel", "arbitrary"),
            vmem_limit_bytes=_VMEM_LIMIT),
        name="l3ap_prep",
    )(z2t, mem_pad, bf(p['mk_w'].T), row(p['mk_b']), bf(p['mv_w'].T),
      row(p['mv_b']), bf(p['in_w'].T), row(p['in_b']), bf(p['mq_w'].T),
      row(p['mq_b']), bf(w0t[:d3]), bf(w0t[d3:]),
      row(p['bih0'] + p['bhh0']))

    bh = bsz // 2
    h1seq = pl.pallas_call(
        _scan_body,
        grid=(2, t),
        in_specs=[pl.BlockSpec((1, bh, h4), lambda c, s: (s, c, 0)),
                  _vspec(), _vspec(), _vspec()],
        out_specs=pl.BlockSpec((1, bh, h), lambda c, s: (s, c, 0)),
        out_shape=jax.ShapeDtypeStruct((t, bsz, h), _F32),
        scratch_shapes=[pltpu.VMEM((bh, h), _BF16),
                        pltpu.VMEM((bh, h), _F32),
                        pltpu.VMEM((bh, h), _BF16),
                        pltpu.VMEM((bh, h), _F32)],
        compiler_params=pltpu.CompilerParams(
            dimension_semantics=("parallel", "arbitrary"),
            vmem_limit_bytes=_VMEM_LIMIT),
        name="l3ap_scan",
    )(xg0, bf(p['whh0'].T), bf(w1cat), row(p['bih1'] + p['bhh1']))

    rows = t * bsz
    rb = 256
    n3 = rows // rb // 2
    hflat = h1seq.reshape(rows, h)
    enc_flat, head_flat = pl.pallas_call(
        _enc_body,
        grid=(2, n3),
        in_specs=[pl.BlockSpec((rb, h), lambda c, j: (c * n3 + j, 0))]
        + [_vspec()] * 12,
        out_specs=[pl.BlockSpec((rb, d3), lambda c, j: (c * n3 + j, 0)),
                   pl.BlockSpec((rb, 128), lambda c, j: (c * n3 + j, 0))],
        out_shape=[jax.ShapeDtypeStruct((rows, d3), _F32),
                   jax.ShapeDtypeStruct((rows, 128), _F32)],
        compiler_params=pltpu.CompilerParams(
            dimension_semantics=("parallel", "arbitrary"),
            vmem_limit_bytes=_VMEM_LIMIT),
        name="l3ap_enc",
    )(hflat, bf(p['e1_w'].T), row(p['e1_b']), row(p['ln1_g']),
      row(p['ln1_b']), bf(p['e2_w'].T), row(p['e2_b']), row(p['ln2_g']),
      row(p['ln2_b']), bf(p['e3_w'].T), row(p['e3_b']), bf(headw.T),
      row(headb))

    zf = enc_flat[rows - bsz:]
    pred = pl.pallas_call(
        _pred_body,
        in_specs=[_vspec()] * 5,
        out_specs=pl.BlockSpec(memory_space=pltpu.VMEM),
        out_shape=jax.ShapeDtypeStruct((bsz, d3), _F32),
        compiler_params=pltpu.CompilerParams(vmem_limit_bytes=_VMEM_LIMIT),
        name="l3ap_pred",
    )(zf, bf(p['p1_w'].T), row(p['p1_b']), bf(p['p2_w'].T), row(p['p2_b']))

    enc = enc_flat.reshape(t, bsz, d3).swapaxes(0, 1)
    heads = head_flat.reshape(t, bsz, 128).swapaxes(0, 1)
    mood = heads[..., :p['mood_w'].shape[0]]
    pers = heads[..., p['mood_w'].shape[0]:
                 p['mood_w'].shape[0] + p['pers_w'].shape[0]]
    return (enc, mood, pers, pred)


# speedup vs baseline: 3.0075x; 1.6691x over previous
import math

import jax
import jax.numpy as jnp
from jax.experimental import pallas as pl
from jax.experimental.pallas import tpu as pltpu

_F32 = jnp.float32
_BF16 = jnp.bfloat16
_NEG = -1e30
_VMEM_LIMIT = 56 * 1024 * 1024


def _sigmoid(x):
    return 1.0 / (1.0 + jnp.exp(-x))


def _tanh(x):
    return 1.0 - 2.0 / (jnp.exp(2.0 * x) + 1.0)


def _erf(x):
    a1, a2, a3 = 0.254829592, -0.284496736, 1.421413741
    a4, a5, p = -1.453152027, 1.061405429, 0.3275911
    ax = jnp.abs(x)
    t = 1.0 / (1.0 + p * ax)
    poly = ((((a5 * t + a4) * t + a3) * t + a2) * t + a1) * t
    y = 1.0 - poly * jnp.exp(-ax * ax)
    return jnp.where(x < 0.0, -y, y)


def _gelu(x):
    return 0.5 * x * (1.0 + _erf(x * (1.0 / math.sqrt(2.0))))


def _ln(x, g, b):
    mu = jnp.mean(x, axis=-1, keepdims=True)
    xc = x - mu
    var = jnp.mean(xc * xc, axis=-1, keepdims=True)
    return xc * jax.lax.rsqrt(var + 1e-5) * g + b


def _dot(a, b):
    return jnp.dot(a, b, preferred_element_type=_F32)


def _make_prep_body(nmem):
    def _prep_body(zt_ref, mem_ref, mkw_ref, mkb_ref, mvw_ref, mvb_ref,
                   inw_ref, inb_ref, mqw_ref, mqb_ref, w0a_ref, w0b_ref,
                   b0_ref, out_ref, keys_scr, vals_scr):
        tb, bsz, d2 = zt_ref.shape
        dm = mem_ref.shape[1]

        @pl.when(pl.program_id(0) == 0)
        def _():
            memb = mem_ref[...].astype(_BF16)
            keys_scr[...] = (_dot(memb, mkw_ref[...])
                             + mkb_ref[...]).astype(_BF16)
            vals_scr[...] = (_dot(memb, mvw_ref[...])
                             + mvb_ref[...]).astype(_BF16)

        x = zt_ref[...].reshape(tb * bsz, d2).astype(_BF16)
        z3 = _dot(x, inw_ref[...]) + inb_ref[...]
        z3b = z3.astype(_BF16)
        q = (_dot(z3b, mqw_ref[...]) + mqb_ref[...]).astype(_BF16)
        s = jax.lax.dot_general(q, keys_scr[...], (((1,), (1,)), ((), ())),
                                preferred_element_type=_F32)
        s = s * (1.0 / math.sqrt(dm))
        lane = jax.lax.broadcasted_iota(jnp.int32, s.shape, 1)
        s = jnp.where(lane < nmem, s, _NEG)
        m = jnp.max(s, axis=-1, keepdims=True)
        e = jnp.exp(s - m)
        attn = (e / jnp.sum(e, axis=-1, keepdims=True)).astype(_BF16)
        read = _dot(attn, vals_scr[...]).astype(_BF16)
        xg = _dot(z3b, w0a_ref[...]) + _dot(read, w0b_ref[...]) + b0_ref[...]
        out_ref[...] = xg.reshape(tb, bsz, out_ref.shape[2])
    return _prep_body


def _scan_body(xg_ref, w0_ref, w1_ref, b1_ref, out_ref,
               h0_scr, c0_scr, h1_scr, c1_scr):
    h = out_ref.shape[2]

    @pl.when(pl.program_id(0) == 0)
    def _():
        h0_scr[...] = jnp.zeros_like(h0_scr)
        c0_scr[...] = jnp.zeros_like(c0_scr)
        h1_scr[...] = jnp.zeros_like(h1_scr)
        c1_scr[...] = jnp.zeros_like(c1_scr)

    g0 = xg_ref[0] + _dot(h0_scr[...], w0_ref[...])
    i0 = _sigmoid(g0[:, :h])
    f0 = _sigmoid(g0[:, h:2 * h])
    gg0 = _tanh(g0[:, 2 * h:3 * h])
    o0 = _sigmoid(g0[:, 3 * h:])
    c0 = f0 * c0_scr[...] + i0 * gg0
    h0b = (o0 * _tanh(c0)).astype(_BF16)
    c0_scr[...] = c0
    h0_scr[...] = h0b

    hcat = jnp.concatenate([h0b, h1_scr[...]], axis=1)
    g1 = _dot(hcat, w1_ref[...]) + b1_ref[...]
    i1 = _sigmoid(g1[:, :h])
    f1 = _sigmoid(g1[:, h:2 * h])
    gg1 = _tanh(g1[:, 2 * h:3 * h])
    o1 = _sigmoid(g1[:, 3 * h:])
    c1 = f1 * c1_scr[...] + i1 * gg1
    h1 = o1 * _tanh(c1)
    c1_scr[...] = c1
    h1_scr[...] = h1.astype(_BF16)
    out_ref[0] = h1


def _enc_body(x_ref, e1w_ref, e1b_ref, g1_ref, bb1_ref, e2w_ref, e2b_ref,
              g2_ref, bb2_ref, e3w_ref, e3b_ref, hw_ref, hb_ref,
              enc_ref, head_ref):
    x = x_ref[...].astype(_BF16)
    h = _gelu(_ln(_dot(x, e1w_ref[...]) + e1b_ref[...],
                  g1_ref[...], bb1_ref[...]))
    h = _gelu(_ln(_dot(h.astype(_BF16), e2w_ref[...]) + e2b_ref[...],
                  g2_ref[...], bb2_ref[...]))
    enc = _dot(h.astype(_BF16), e3w_ref[...]) + e3b_ref[...]
    enc_ref[...] = enc
    hd = _dot(enc.astype(_BF16), hw_ref[...]) + hb_ref[...]
    lane = jax.lax.broadcasted_iota(jnp.int32, hd.shape, 1)
    head_ref[...] = jnp.where(lane < 3, hd, _sigmoid(hd))


def _pred_body(zf_ref, p1w_ref, p1b_ref, p2w_ref, p2b_ref, out_ref):
    h = _gelu(_dot(zf_ref[...].astype(_BF16), p1w_ref[...]) + p1b_ref[...])
    out_ref[...] = _dot(h.astype(_BF16), p2w_ref[...]) + p2b_ref[...]


def _vspec():
    return pl.BlockSpec(memory_space=pltpu.VMEM)


def kernel(z2, params):
    p = params
    bsz, t, d2 = z2.shape
    d3 = p['in_w'].shape[0]
    dm = p['mq_w'].shape[0]
    h4 = p['wih0'].shape[0]
    h = h4 // 4
    de1 = p['e1_w'].shape[0]
    de2 = p['e2_w'].shape[0]

    def bf(x):
        return x.astype(_BF16)

    def row(x):
        return x.reshape(1, -1)

    z2t = jnp.swapaxes(z2, 0, 1)
    mem = p['memory'][0]
    mem_pad = jnp.pad(mem, ((0, 128 - mem.shape[0]), (0, 0)))
    w0t = p['wih0'].T
    w1cat = jnp.concatenate([p['wih1'].T, p['whh1'].T], axis=0)
    headw = jnp.concatenate(
        [p['mood_w'], p['pers_w'],
         jnp.zeros((128 - p['mood_w'].shape[0] - p['pers_w'].shape[0], d3),
                   _F32)], axis=0)
    headb = jnp.concatenate(
        [p['mood_b'], p['pers_b'],
         jnp.zeros((128 - p['mood_b'].shape[0] - p['pers_b'].shape[0],),
                   _F32)], axis=0)

    tb = 32
    n1 = t // tb
    xg0 = pl.pallas_call(
        _make_prep_body(mem.shape[0]),
        grid=(n1,),
        in_specs=[pl.BlockSpec((tb, bsz, d2), lambda j: (j, 0, 0))]
        + [_vspec()] * 12,
        out_specs=pl.BlockSpec((tb, bsz, h4), lambda j: (j, 0, 0)),
        out_shape=jax.ShapeDtypeStruct((t, bsz, h4), _F32),
        scratch_shapes=[pltpu.VMEM((128, dm), _BF16),
                        pltpu.VMEM((128, dm), _BF16)],
        compiler_params=pltpu.CompilerParams(
            dimension_semantics=("arbitrary",),
            vmem_limit_bytes=_VMEM_LIMIT),
        name="l3ap_prep",
    )(z2t, mem_pad, bf(p['mk_w'].T), row(p['mk_b']), bf(p['mv_w'].T),
      row(p['mv_b']), bf(p['in_w'].T), row(p['in_b']), bf(p['mq_w'].T),
      row(p['mq_b']), bf(w0t[:d3]), bf(w0t[d3:]),
      row(p['bih0'] + p['bhh0']))

    h1seq = pl.pallas_call(
        _scan_body,
        grid=(t,),
        in_specs=[pl.BlockSpec((1, bsz, h4), lambda s: (s, 0, 0)),
                  _vspec(), _vspec(), _vspec()],
        out_specs=pl.BlockSpec((1, bsz, h), lambda s: (s, 0, 0)),
        out_shape=jax.ShapeDtypeStruct((t, bsz, h), _F32),
        scratch_shapes=[pltpu.VMEM((bsz, h), _BF16),
                        pltpu.VMEM((bsz, h), _F32),
                        pltpu.VMEM((bsz, h), _BF16),
                        pltpu.VMEM((bsz, h), _F32)],
        compiler_params=pltpu.CompilerParams(
            dimension_semantics=("arbitrary",),
            vmem_limit_bytes=_VMEM_LIMIT),
        name="l3ap_scan",
    )(xg0, bf(p['whh0'].T), bf(w1cat), row(p['bih1'] + p['bhh1']))

    rows = t * bsz
    rb = 256
    n3 = rows // rb
    hflat = h1seq.reshape(rows, h)
    enc_flat, head_flat = pl.pallas_call(
        _enc_body,
        grid=(n3,),
        in_specs=[pl.BlockSpec((rb, h), lambda j: (j, 0))]
        + [_vspec()] * 12,
        out_specs=[pl.BlockSpec((rb, d3), lambda j: (j, 0)),
                   pl.BlockSpec((rb, 128), lambda j: (j, 0))],
        out_shape=[jax.ShapeDtypeStruct((rows, d3), _F32),
                   jax.ShapeDtypeStruct((rows, 128), _F32)],
        compiler_params=pltpu.CompilerParams(
            dimension_semantics=("arbitrary",),
            vmem_limit_bytes=_VMEM_LIMIT),
        name="l3ap_enc",
    )(hflat, bf(p['e1_w'].T), row(p['e1_b']), row(p['ln1_g']),
      row(p['ln1_b']), bf(p['e2_w'].T), row(p['e2_b']), row(p['ln2_g']),
      row(p['ln2_b']), bf(p['e3_w'].T), row(p['e3_b']), bf(headw.T),
      row(headb))

    zf = enc_flat[rows - bsz:]
    pred = pl.pallas_call(
        _pred_body,
        in_specs=[_vspec()] * 5,
        out_specs=pl.BlockSpec(memory_space=pltpu.VMEM),
        out_shape=jax.ShapeDtypeStruct((bsz, d3), _F32),
        compiler_params=pltpu.CompilerParams(vmem_limit_bytes=_VMEM_LIMIT),
        name="l3ap_pred",
    )(zf, bf(p['p1_w'].T), row(p['p1_b']), bf(p['p2_w'].T), row(p['p2_b']))

    enc = enc_flat.reshape(t, bsz, d3).swapaxes(0, 1)
    heads = head_flat.reshape(t, bsz, 128).swapaxes(0, 1)
    mood = heads[..., :p['mood_w'].shape[0]]
    pers = heads[..., p['mood_w'].shape[0]:
                 p['mood_w'].shape[0] + p['pers_w'].shape[0]]
    return (enc, mood, pers, pred)


# speedup vs baseline: 3.0924x; 1.0282x over previous
import math

import jax
import jax.numpy as jnp
from jax.experimental import pallas as pl
from jax.experimental.pallas import tpu as pltpu

_F32 = jnp.float32
_BF16 = jnp.bfloat16
_NEG = -1e30
_VMEM_LIMIT = 56 * 1024 * 1024


def _sigmoid(x):
    return 1.0 / (1.0 + jnp.exp(-x))


def _tanh(x):
    return 1.0 - 2.0 / (jnp.exp(2.0 * x) + 1.0)


def _erf(x):
    a1, a2, a3 = 0.254829592, -0.284496736, 1.421413741
    a4, a5, p = -1.453152027, 1.061405429, 0.3275911
    ax = jnp.abs(x)
    t = 1.0 / (1.0 + p * ax)
    poly = ((((a5 * t + a4) * t + a3) * t + a2) * t + a1) * t
    y = 1.0 - poly * jnp.exp(-ax * ax)
    return jnp.where(x < 0.0, -y, y)


def _gelu(x):
    return 0.5 * x * (1.0 + _erf(x * (1.0 / math.sqrt(2.0))))


def _ln(x, g, b):
    mu = jnp.mean(x, axis=-1, keepdims=True)
    xc = x - mu
    var = jnp.mean(xc * xc, axis=-1, keepdims=True)
    return xc * jax.lax.rsqrt(var + 1e-5) * g + b


def _dot(a, b):
    return jnp.dot(a, b, preferred_element_type=_F32)


def _make_prep_body(nmem):
    def _prep_body(zt_ref, mem_ref, mkw_ref, mkb_ref, mvw_ref, mvb_ref,
                   inw_ref, inb_ref, mqw_ref, mqb_ref, w0a_ref, w0b_ref,
                   b0_ref, out_ref, keys_scr, vals_scr):
        bsz, tb, d2 = zt_ref.shape
        dm = mem_ref.shape[1]

        @pl.when(pl.program_id(0) == 0)
        def _():
            memb = mem_ref[...].astype(_BF16)
            keys_scr[...] = (_dot(memb, mkw_ref[...])
                             + mkb_ref[...]).astype(_BF16)
            vals_scr[...] = (_dot(memb, mvw_ref[...])
                             + mvb_ref[...]).astype(_BF16)

        x = jnp.swapaxes(zt_ref[...], 0, 1).reshape(tb * bsz, d2)
        x = x.astype(_BF16)
        z3 = _dot(x, inw_ref[...]) + inb_ref[...]
        z3b = z3.astype(_BF16)
        q = (_dot(z3b, mqw_ref[...]) + mqb_ref[...]).astype(_BF16)
        s = jax.lax.dot_general(q, keys_scr[...], (((1,), (1,)), ((), ())),
                                preferred_element_type=_F32)
        s = s * (1.0 / math.sqrt(dm))
        lane = jax.lax.broadcasted_iota(jnp.int32, s.shape, 1)
        s = jnp.where(lane < nmem, s, _NEG)
        m = jnp.max(s, axis=-1, keepdims=True)
        e = jnp.exp(s - m)
        attn = (e / jnp.sum(e, axis=-1, keepdims=True)).astype(_BF16)
        read = _dot(attn, vals_scr[...]).astype(_BF16)
        xg = _dot(z3b, w0a_ref[...]) + _dot(read, w0b_ref[...]) + b0_ref[...]
        out_ref[...] = xg.reshape(tb, bsz, out_ref.shape[2])
    return _prep_body


def _scan_body(xg_ref, w0_ref, w1_ref, b1_ref, out_ref,
               h0_scr, c0_scr, h1_scr, c1_scr):
    h = out_ref.shape[2]

    @pl.when(pl.program_id(0) == 0)
    def _():
        h0_scr[...] = jnp.zeros_like(h0_scr)
        c0_scr[...] = jnp.zeros_like(c0_scr)
        h1_scr[...] = jnp.zeros_like(h1_scr)
        c1_scr[...] = jnp.zeros_like(c1_scr)

    h0b = h0_scr[...]
    c0 = c0_scr[...]
    h1b = h1_scr[...]
    c1 = c1_scr[...]
    for st in range(xg_ref.shape[0]):
        g0 = xg_ref[st] + _dot(h0b, w0_ref[...])
        i0 = _sigmoid(g0[:, :h])
        f0 = _sigmoid(g0[:, h:2 * h])
        gg0 = _tanh(g0[:, 2 * h:3 * h])
        o0 = _sigmoid(g0[:, 3 * h:])
        c0 = f0 * c0 + i0 * gg0
        h0b = (o0 * _tanh(c0)).astype(_BF16)

        hcat = jnp.concatenate([h0b, h1b], axis=1)
        g1 = _dot(hcat, w1_ref[...]) + b1_ref[...]
        i1 = _sigmoid(g1[:, :h])
        f1 = _sigmoid(g1[:, h:2 * h])
        gg1 = _tanh(g1[:, 2 * h:3 * h])
        o1 = _sigmoid(g1[:, 3 * h:])
        c1 = f1 * c1 + i1 * gg1
        h1 = o1 * _tanh(c1)
        h1b = h1.astype(_BF16)
        out_ref[st] = h1
    h0_scr[...] = h0b
    c0_scr[...] = c0
    h1_scr[...] = h1b
    c1_scr[...] = c1


def _make_enc_body(nlast):
    def _enc_body(x_ref, e1w_ref, e1b_ref, g1_ref, bb1_ref, e2w_ref,
                  e2b_ref, g2_ref, bb2_ref, e3w_ref, e3b_ref, hw_ref,
                  hb_ref, p1w_ref, p1b_ref, p2w_ref, p2b_ref,
                  enc_ref, head_ref, pred_ref):
        rb = x_ref.shape[0]
        nb = enc_ref.shape[0]
        nt = enc_ref.shape[1]
        x = x_ref[...].astype(_BF16)
        hh = _gelu(_ln(_dot(x, e1w_ref[...]) + e1b_ref[...],
                       g1_ref[...], bb1_ref[...]))
        hh = _gelu(_ln(_dot(hh.astype(_BF16), e2w_ref[...]) + e2b_ref[...],
                       g2_ref[...], bb2_ref[...]))
        enc = _dot(hh.astype(_BF16), e3w_ref[...]) + e3b_ref[...]
        enc_ref[...] = jnp.swapaxes(enc.reshape(nt, nb, enc.shape[1]), 0, 1)
        hd = _dot(enc.astype(_BF16), hw_ref[...]) + hb_ref[...]
        lane = jax.lax.broadcasted_iota(jnp.int32, hd.shape, 1)
        hd = jnp.where(lane < 3, hd, _sigmoid(hd))
        head_ref[...] = jnp.swapaxes(hd.reshape(nt, nb, hd.shape[1]), 0, 1)

        @pl.when(pl.program_id(0) == nlast - 1)
        def _():
            zf = enc[rb - nb:, :].astype(_BF16)
            hp = _gelu(_dot(zf, p1w_ref[...]) + p1b_ref[...])
            pred_ref[...] = _dot(hp.astype(_BF16), p2w_ref[...]) + p2b_ref[...]
    return _enc_body


def _vspec():
    return pl.BlockSpec(memory_space=pltpu.VMEM)


def kernel(z2, params):
    p = params
    bsz, t, d2 = z2.shape
    d3 = p['in_w'].shape[0]
    dm = p['mq_w'].shape[0]
    h4 = p['wih0'].shape[0]
    h = h4 // 4

    def bf(x):
        return x.astype(_BF16)

    def row(x):
        return x.reshape(1, -1)

    mem = p['memory'][0]
    mem_pad = jnp.pad(mem, ((0, 128 - mem.shape[0]), (0, 0)))
    w0t = p['wih0'].T
    w1cat = jnp.concatenate([p['wih1'].T, p['whh1'].T], axis=0)
    headw = jnp.concatenate(
        [p['mood_w'], p['pers_w'],
         jnp.zeros((128 - p['mood_w'].shape[0] - p['pers_w'].shape[0], d3),
                   _F32)], axis=0)
    headb = jnp.concatenate(
        [p['mood_b'], p['pers_b'],
         jnp.zeros((128 - p['mood_b'].shape[0] - p['pers_b'].shape[0],),
                   _F32)], axis=0)

    tb = 32
    n1 = t // tb
    xg0 = pl.pallas_call(
        _make_prep_body(mem.shape[0]),
        grid=(n1,),
        in_specs=[pl.BlockSpec((bsz, tb, d2), lambda j: (0, j, 0))]
        + [_vspec()] * 12,
        out_specs=pl.BlockSpec((tb, bsz, h4), lambda j: (j, 0, 0)),
        out_shape=jax.ShapeDtypeStruct((t, bsz, h4), _F32),
        scratch_shapes=[pltpu.VMEM((128, dm), _BF16),
                        pltpu.VMEM((128, dm), _BF16)],
        compiler_params=pltpu.CompilerParams(
            dimension_semantics=("arbitrary",),
            vmem_limit_bytes=_VMEM_LIMIT),
        name="l3ap_prep",
    )(z2, mem_pad, bf(p['mk_w'].T), row(p['mk_b']), bf(p['mv_w'].T),
      row(p['mv_b']), bf(p['in_w'].T), row(p['in_b']), bf(p['mq_w'].T),
      row(p['mq_b']), bf(w0t[:d3]), bf(w0t[d3:]),
      row(p['bih0'] + p['bhh0']))

    ts = 2
    h1seq = pl.pallas_call(
        _scan_body,
        grid=(t // ts,),
        in_specs=[pl.BlockSpec((ts, bsz, h4), lambda s: (s, 0, 0)),
                  _vspec(), _vspec(), _vspec()],
        out_specs=pl.BlockSpec((ts, bsz, h), lambda s: (s, 0, 0)),
        out_shape=jax.ShapeDtypeStruct((t, bsz, h), _F32),
        scratch_shapes=[pltpu.VMEM((bsz, h), _BF16),
                        pltpu.VMEM((bsz, h), _F32),
                        pltpu.VMEM((bsz, h), _BF16),
                        pltpu.VMEM((bsz, h), _F32)],
        compiler_params=pltpu.CompilerParams(
            dimension_semantics=("arbitrary",),
            vmem_limit_bytes=_VMEM_LIMIT),
        name="l3ap_scan",
    )(xg0, bf(p['whh0'].T), bf(w1cat), row(p['bih1'] + p['bhh1']))

    rows = t * bsz
    rb = 256
    n3 = rows // rb
    nt = rb // bsz
    hflat = h1seq.reshape(rows, h)
    enc, heads, pred = pl.pallas_call(
        _make_enc_body(n3),
        grid=(n3,),
        in_specs=[pl.BlockSpec((rb, h), lambda j: (j, 0))]
        + [_vspec()] * 16,
        out_specs=[pl.BlockSpec((bsz, nt, d3), lambda j: (0, j, 0)),
                   pl.BlockSpec((bsz, nt, 128), lambda j: (0, j, 0)),
                   pl.BlockSpec((bsz, d3), lambda j: (0, 0))],
        out_shape=[jax.ShapeDtypeStruct((bsz, t, d3), _F32),
                   jax.ShapeDtypeStruct((bsz, t, 128), _F32),
                   jax.ShapeDtypeStruct((bsz, d3), _F32)],
        compiler_params=pltpu.CompilerParams(
            dimension_semantics=("arbitrary",),
            vmem_limit_bytes=_VMEM_LIMIT),
        name="l3ap_enc",
    )(hflat, bf(p['e1_w'].T), row(p['e1_b']), row(p['ln1_g']),
      row(p['ln1_b']), bf(p['e2_w'].T), row(p['e2_b']), row(p['ln2_g']),
      row(p['ln2_b']), bf(p['e3_w'].T), row(p['e3_b']), bf(headw.T),
      row(headb), bf(p['p1_w'].T), row(p['p1_b']), bf(p['p2_w'].T),
      row(p['p2_b']))

    mood = heads[..., :p['mood_w'].shape[0]]
    pers = heads[..., p['mood_w'].shape[0]:
                 p['mood_w'].shape[0] + p['pers_w'].shape[0]]
    return (enc, mood, pers, pred)


# speedup vs baseline: 3.1141x; 1.0070x over previous
import math

import jax
import jax.numpy as jnp
from jax.experimental import pallas as pl
from jax.experimental.pallas import tpu as pltpu

_F32 = jnp.float32
_BF16 = jnp.bfloat16
_NEG = -1e30
_VMEM_LIMIT = 56 * 1024 * 1024


def _sigmoid(x):
    return 1.0 / (1.0 + jnp.exp(-x))


def _tanh(x):
    return 1.0 - 2.0 / (jnp.exp(2.0 * x) + 1.0)


def _erf(x):
    a1, a2, a3 = 0.254829592, -0.284496736, 1.421413741
    a4, a5, p = -1.453152027, 1.061405429, 0.3275911
    ax = jnp.abs(x)
    t = 1.0 / (1.0 + p * ax)
    poly = ((((a5 * t + a4) * t + a3) * t + a2) * t + a1) * t
    y = 1.0 - poly * jnp.exp(-ax * ax)
    return jnp.where(x < 0.0, -y, y)


def _gelu(x):
    return 0.5 * x * (1.0 + _erf(x * (1.0 / math.sqrt(2.0))))


def _ln(x, g, b):
    mu = jnp.mean(x, axis=-1, keepdims=True)
    xc = x - mu
    var = jnp.mean(xc * xc, axis=-1, keepdims=True)
    return xc * jax.lax.rsqrt(var + 1e-5) * g + b


def _dot(a, b):
    return jnp.dot(a, b, preferred_element_type=_F32)


def _dot_t(a, b):
    return jax.lax.dot_general(a, b, (((1,), (1,)), ((), ())),
                               preferred_element_type=_F32)


def _make_prep_body(nmem):
    def _prep_body(zt_ref, mem_ref, mkw_ref, mkb_ref, mvw_ref, mvb_ref,
                   inw_ref, inb_ref, mqw_ref, mqb_ref, w0a_ref, w0b_ref,
                   b0_ref, out_ref, keys_scr, vals_scr):
        bsz, tb, d2 = zt_ref.shape
        dm = mem_ref.shape[1]

        @pl.when(pl.program_id(0) == 0)
        def _():
            memb = mem_ref[...].astype(_BF16)
            keys_scr[...] = (_dot(memb, mkw_ref[...])
                             + mkb_ref[...]).astype(_BF16)
            vals_scr[...] = (_dot(memb, mvw_ref[...])
                             + mvb_ref[...]).astype(_BF16)

        x = jnp.swapaxes(zt_ref[...], 0, 1).reshape(tb * bsz, d2)
        x = x.astype(_BF16)
        z3 = _dot(x, inw_ref[...]) + inb_ref[...]
        z3b = z3.astype(_BF16)
        q = (_dot(z3b, mqw_ref[...]) + mqb_ref[...]).astype(_BF16)
        s = _dot_t(q, keys_scr[...]) * (1.0 / math.sqrt(dm))
        lane = jax.lax.broadcasted_iota(jnp.int32, s.shape, 1)
        s = jnp.where(lane < nmem, s, _NEG)
        m = jnp.max(s, axis=-1, keepdims=True)
        e = jnp.exp(s - m)
        attn = (e / jnp.sum(e, axis=-1, keepdims=True)).astype(_BF16)
        read = _dot(attn, vals_scr[...]).astype(_BF16)
        xg = (_dot_t(z3b, w0a_ref[...]) + _dot_t(read, w0b_ref[...])
              + b0_ref[...])
        out_ref[...] = xg.reshape(tb, bsz, out_ref.shape[2])
    return _prep_body


def _scan_body(xg_ref, w0_ref, w1_ref, b1_ref, out_ref,
               h0_scr, c0_scr, h1_scr, c1_scr):
    h = out_ref.shape[2]

    @pl.when(pl.program_id(0) == 0)
    def _():
        h0_scr[...] = jnp.zeros_like(h0_scr)
        c0_scr[...] = jnp.zeros_like(c0_scr)
        h1_scr[...] = jnp.zeros_like(h1_scr)
        c1_scr[...] = jnp.zeros_like(c1_scr)

    h0b = h0_scr[...]
    c0 = c0_scr[...]
    h1b = h1_scr[...]
    c1 = c1_scr[...]
    for st in range(xg_ref.shape[0]):
        g0 = xg_ref[st] + _dot(h0b, w0_ref[...])
        i0 = _sigmoid(g0[:, :h])
        f0 = _sigmoid(g0[:, h:2 * h])
        gg0 = _tanh(g0[:, 2 * h:3 * h])
        o0 = _sigmoid(g0[:, 3 * h:])
        c0 = f0 * c0 + i0 * gg0
        h0b = (o0 * _tanh(c0)).astype(_BF16)

        hcat = jnp.concatenate([h0b, h1b], axis=1)
        g1 = _dot(hcat, w1_ref[...]) + b1_ref[...]
        i1 = _sigmoid(g1[:, :h])
        f1 = _sigmoid(g1[:, h:2 * h])
        gg1 = _tanh(g1[:, 2 * h:3 * h])
        o1 = _sigmoid(g1[:, 3 * h:])
        c1 = f1 * c1 + i1 * gg1
        h1 = o1 * _tanh(c1)
        h1b = h1.astype(_BF16)
        out_ref[st] = h1
    h0_scr[...] = h0b
    c0_scr[...] = c0
    h1_scr[...] = h1b
    c1_scr[...] = c1


def _make_enc_body(nlast):
    def _enc_body(x_ref, e1w_ref, e1b_ref, g1_ref, bb1_ref, e2w_ref,
                  e2b_ref, g2_ref, bb2_ref, e3w_ref, e3b_ref, hw_ref,
                  hb_ref, p1w_ref, p1b_ref, p2w_ref, p2b_ref,
                  enc_ref, head_ref, pred_ref):
        rb = x_ref.shape[0]
        nb = enc_ref.shape[0]
        nt = enc_ref.shape[1]
        x = x_ref[...].astype(_BF16)
        hh = _gelu(_ln(_dot_t(x, e1w_ref[...]) + e1b_ref[...],
                       g1_ref[...], bb1_ref[...]))
        hh = _gelu(_ln(_dot_t(hh.astype(_BF16), e2w_ref[...]) + e2b_ref[...],
                       g2_ref[...], bb2_ref[...]))
        enc = _dot_t(hh.astype(_BF16), e3w_ref[...]) + e3b_ref[...]
        enc_ref[...] = jnp.swapaxes(enc.reshape(nt, nb, enc.shape[1]), 0, 1)
        hd = _dot_t(enc.astype(_BF16), hw_ref[...]) + hb_ref[...]
        lane = jax.lax.broadcasted_iota(jnp.int32, hd.shape, 1)
        hd = jnp.where(lane < 3, hd, _sigmoid(hd))
        head_ref[...] = jnp.swapaxes(hd.reshape(nt, nb, hd.shape[1]), 0, 1)

        @pl.when(pl.program_id(0) == nlast - 1)
        def _():
            zf = enc[rb - nb:, :].astype(_BF16)
            hp = _gelu(_dot_t(zf, p1w_ref[...]) + p1b_ref[...])
            pred_ref[...] = (_dot_t(hp.astype(_BF16), p2w_ref[...])
                             + p2b_ref[...])
    return _enc_body


def _vspec():
    return pl.BlockSpec(memory_space=pltpu.VMEM)


def kernel(z2, params):
    p = params
    bsz, t, d2 = z2.shape
    d3 = p['in_w'].shape[0]
    dm = p['mq_w'].shape[0]
    h4 = p['wih0'].shape[0]
    h = h4 // 4

    def bf(x):
        return x.astype(_BF16)

    def row(x):
        return x.reshape(1, -1)

    mem = p['memory'][0]
    mem_pad = jnp.pad(mem, ((0, 128 - mem.shape[0]), (0, 0)))
    w1cat = jnp.concatenate([p['wih1'].astype(_BF16).T,
                             p['whh1'].astype(_BF16).T], axis=0)
    headw = jnp.concatenate(
        [p['mood_w'], p['pers_w'],
         jnp.zeros((128 - p['mood_w'].shape[0] - p['pers_w'].shape[0], d3),
                   _F32)], axis=0)
    headb = jnp.concatenate(
        [p['mood_b'], p['pers_b'],
         jnp.zeros((128 - p['mood_b'].shape[0] - p['pers_b'].shape[0],),
                   _F32)], axis=0)

    tb = 32
    n1 = t // tb
    xg0 = pl.pallas_call(
        _make_prep_body(mem.shape[0]),
        grid=(n1,),
        in_specs=[pl.BlockSpec((bsz, tb, d2), lambda j: (0, j, 0))]
        + [_vspec()] * 12,
        out_specs=pl.BlockSpec((tb, bsz, h4), lambda j: (j, 0, 0)),
        out_shape=jax.ShapeDtypeStruct((t, bsz, h4), _F32),
        scratch_shapes=[pltpu.VMEM((128, dm), _BF16),
                        pltpu.VMEM((128, dm), _BF16)],
        compiler_params=pltpu.CompilerParams(
            dimension_semantics=("arbitrary",),
            vmem_limit_bytes=_VMEM_LIMIT),
        name="l3ap_prep",
    )(z2, mem_pad, bf(p['mk_w'].T), row(p['mk_b']), bf(p['mv_w'].T),
      row(p['mv_b']), bf(p['in_w'].T), row(p['in_b']), bf(p['mq_w'].T),
      row(p['mq_b']), bf(p['wih0'][:, :d3]), bf(p['wih0'][:, d3:]),
      row(p['bih0'] + p['bhh0']))

    ts = 8
    h1seq = pl.pallas_call(
        _scan_body,
        grid=(t // ts,),
        in_specs=[pl.BlockSpec((ts, bsz, h4), lambda s: (s, 0, 0)),
                  _vspec(), _vspec(), _vspec()],
        out_specs=pl.BlockSpec((ts, bsz, h), lambda s: (s, 0, 0)),
        out_shape=jax.ShapeDtypeStruct((t, bsz, h), _F32),
        scratch_shapes=[pltpu.VMEM((bsz, h), _BF16),
                        pltpu.VMEM((bsz, h), _F32),
                        pltpu.VMEM((bsz, h), _BF16),
                        pltpu.VMEM((bsz, h), _F32)],
        compiler_params=pltpu.CompilerParams(
            dimension_semantics=("arbitrary",),
            vmem_limit_bytes=_VMEM_LIMIT),
        name="l3ap_scan",
    )(xg0, p['whh0'].astype(_BF16).T, w1cat, row(p['bih1'] + p['bhh1']))

    rows = t * bsz
    rb = 256
    n3 = rows // rb
    nt = rb // bsz
    hflat = h1seq.reshape(rows, h)
    enc, heads, pred = pl.pallas_call(
        _make_enc_body(n3),
        grid=(n3,),
        in_specs=[pl.BlockSpec((rb, h), lambda j: (j, 0))]
        + [_vspec()] * 16,
        out_specs=[pl.BlockSpec((bsz, nt, d3), lambda j: (0, j, 0)),
                   pl.BlockSpec((bsz, nt, 128), lambda j: (0, j, 0)),
                   pl.BlockSpec((bsz, d3), lambda j: (0, 0))],
        out_shape=[jax.ShapeDtypeStruct((bsz, t, d3), _F32),
                   jax.ShapeDtypeStruct((bsz, t, 128), _F32),
                   jax.ShapeDtypeStruct((bsz, d3), _F32)],
        compiler_params=pltpu.CompilerParams(
            dimension_semantics=("arbitrary",),
            vmem_limit_bytes=_VMEM_LIMIT),
        name="l3ap_enc",
    )(hflat, bf(p['e1_w']), row(p['e1_b']), row(p['ln1_g']),
      row(p['ln1_b']), bf(p['e2_w']), row(p['e2_b']), row(p['ln2_g']),
      row(p['ln2_b']), bf(p['e3_w']), row(p['e3_b']), bf(headw),
      row(headb), bf(p['p1_w']), row(p['p1_b']), bf(p['p2_w']),
      row(p['p2_b']))

    mood = heads[..., :p['mood_w'].shape[0]]
    pers = heads[..., p['mood_w'].shape[0]:
                 p['mood_w'].shape[0] + p['pers_w'].shape[0]]
    return (enc, mood, pers, pred)


# speedup vs baseline: 3.8384x; 1.2326x over previous
import math

import jax
import jax.numpy as jnp
from jax.experimental import pallas as pl
from jax.experimental.pallas import tpu as pltpu

_F32 = jnp.float32
_BF16 = jnp.bfloat16
_NEG = -1e30
_VMEM_LIMIT = 56 * 1024 * 1024


def _sigmoid(x):
    return 1.0 / (1.0 + jnp.exp(-x))


def _tanh(x):
    return 1.0 - 2.0 / (jnp.exp(2.0 * x) + 1.0)


def _erf(x):
    a1, a2, a3 = 0.254829592, -0.284496736, 1.421413741
    a4, a5, p = -1.453152027, 1.061405429, 0.3275911
    ax = jnp.abs(x)
    t = 1.0 / (1.0 + p * ax)
    poly = ((((a5 * t + a4) * t + a3) * t + a2) * t + a1) * t
    y = 1.0 - poly * jnp.exp(-ax * ax)
    return jnp.where(x < 0.0, -y, y)


def _gelu(x):
    return 0.5 * x * (1.0 + _erf(x * (1.0 / math.sqrt(2.0))))


def _ln(x, g, b):
    mu = jnp.mean(x, axis=-1, keepdims=True)
    xc = x - mu
    var = jnp.mean(xc * xc, axis=-1, keepdims=True)
    return xc * jax.lax.rsqrt(var + 1e-5) * g + b


def _dot(a, b):
    return jnp.dot(a, b, preferred_element_type=_F32)


def _dot_t(a, b):
    return jax.lax.dot_general(a, b, (((1,), (1,)), ((), ())),
                               preferred_element_type=_F32)


def _make_prep_body(nmem):
    def _prep_body(zt_ref, mem_ref, mkw_ref, mkb_ref, mvw_ref, mvb_ref,
                   inw_ref, inb_ref, mqw_ref, mqb_ref, w0a_ref, w0b_ref,
                   b0_ref, out_ref, keys_scr, vals_scr):
        bsz, tb, d2 = zt_ref.shape
        dm = mem_ref.shape[1]

        @pl.when(pl.program_id(0) == 0)
        def _():
            memb = mem_ref[...].astype(_BF16)
            keys_scr[...] = (_dot(memb, mkw_ref[...])
                             + mkb_ref[...]).astype(_BF16)
            vals_scr[...] = (_dot(memb, mvw_ref[...])
                             + mvb_ref[...]).astype(_BF16)

        x = jnp.swapaxes(zt_ref[...], 0, 1).reshape(tb * bsz, d2)
        x = x.astype(_BF16)
        z3 = _dot(x, inw_ref[...]) + inb_ref[...]
        z3b = z3.astype(_BF16)
        q = (_dot(z3b, mqw_ref[...]) + mqb_ref[...]).astype(_BF16)
        s = _dot_t(q, keys_scr[...]) * (1.0 / math.sqrt(dm))
        lane = jax.lax.broadcasted_iota(jnp.int32, s.shape, 1)
        s = jnp.where(lane < nmem, s, _NEG)
        m = jnp.max(s, axis=-1, keepdims=True)
        e = jnp.exp(s - m)
        attn = (e / jnp.sum(e, axis=-1, keepdims=True)).astype(_BF16)
        read = _dot(attn, vals_scr[...]).astype(_BF16)
        xg = (_dot_t(z3b, w0a_ref[...]) + _dot_t(read, w0b_ref[...])
              + b0_ref[...])
        out_ref[...] = xg.reshape(tb, bsz, out_ref.shape[2])
    return _prep_body


def _scan_body(xg_ref, w0_ref, wih1_ref, w1h_ref, b1_ref, out_ref,
               h0_scr, c0_scr, h1_scr, c1_scr, h0buf, xg1buf):
    ts = xg_ref.shape[0]
    h = out_ref.shape[2]
    bsz = out_ref.shape[1]

    @pl.when(pl.program_id(0) == 0)
    def _():
        h0_scr[...] = jnp.zeros_like(h0_scr)
        c0_scr[...] = jnp.zeros_like(c0_scr)
        h1_scr[...] = jnp.zeros_like(h1_scr)
        c1_scr[...] = jnp.zeros_like(c1_scr)

    h0b = h0_scr[...]
    c0 = c0_scr[...]
    for st in range(ts):
        g0 = xg_ref[st] + _dot(h0b, w0_ref[...])
        i0 = _sigmoid(g0[:, :h])
        f0 = _sigmoid(g0[:, h:2 * h])
        gg0 = _tanh(g0[:, 2 * h:3 * h])
        o0 = _sigmoid(g0[:, 3 * h:])
        c0 = f0 * c0 + i0 * gg0
        h0b = (o0 * _tanh(c0)).astype(_BF16)
        h0buf[st * bsz:(st + 1) * bsz] = h0b
    h0_scr[...] = h0b
    c0_scr[...] = c0

    xg1buf[...] = _dot_t(h0buf[...], wih1_ref[...]) + b1_ref[...]

    h1b = h1_scr[...]
    c1 = c1_scr[...]
    for st in range(ts):
        g1 = xg1buf[st * bsz:(st + 1) * bsz] + _dot(h1b, w1h_ref[...])
        i1 = _sigmoid(g1[:, :h])
        f1 = _sigmoid(g1[:, h:2 * h])
        gg1 = _tanh(g1[:, 2 * h:3 * h])
        o1 = _sigmoid(g1[:, 3 * h:])
        c1 = f1 * c1 + i1 * gg1
        h1 = o1 * _tanh(c1)
        h1b = h1.astype(_BF16)
        out_ref[st] = h1
    h1_scr[...] = h1b
    c1_scr[...] = c1


def _make_enc_body(nlast):
    def _enc_body(x_ref, e1w_ref, e1b_ref, g1_ref, bb1_ref, e2w_ref,
                  e2b_ref, g2_ref, bb2_ref, e3w_ref, e3b_ref, hw_ref,
                  hb_ref, p1w_ref, p1b_ref, p2w_ref, p2b_ref,
                  enc_ref, head_ref, pred_ref):
        rb = x_ref.shape[0]
        nb = enc_ref.shape[0]
        nt = enc_ref.shape[1]
        x = x_ref[...].astype(_BF16)
        hh = _gelu(_ln(_dot_t(x, e1w_ref[...]) + e1b_ref[...],
                       g1_ref[...], bb1_ref[...]))
        hh = _gelu(_ln(_dot_t(hh.astype(_BF16), e2w_ref[...]) + e2b_ref[...],
                       g2_ref[...], bb2_ref[...]))
        enc = _dot_t(hh.astype(_BF16), e3w_ref[...]) + e3b_ref[...]
        enc_ref[...] = jnp.swapaxes(enc.reshape(nt, nb, enc.shape[1]), 0, 1)
        hd = _dot_t(enc.astype(_BF16), hw_ref[...]) + hb_ref[...]
        lane = jax.lax.broadcasted_iota(jnp.int32, hd.shape, 1)
        hd = jnp.where(lane < 3, hd, _sigmoid(hd))
        head_ref[...] = jnp.swapaxes(hd.reshape(nt, nb, hd.shape[1]), 0, 1)

        @pl.when(pl.program_id(0) == nlast - 1)
        def _():
            zf = enc[rb - nb:, :].astype(_BF16)
            hp = _gelu(_dot_t(zf, p1w_ref[...]) + p1b_ref[...])
            pred_ref[...] = (_dot_t(hp.astype(_BF16), p2w_ref[...])
                             + p2b_ref[...])
    return _enc_body


def _vspec():
    return pl.BlockSpec(memory_space=pltpu.VMEM)


def kernel(z2, params):
    p = params
    bsz, t, d2 = z2.shape
    d3 = p['in_w'].shape[0]
    dm = p['mq_w'].shape[0]
    h4 = p['wih0'].shape[0]
    h = h4 // 4

    def bf(x):
        return x.astype(_BF16)

    def row(x):
        return x.reshape(1, -1)

    mem = p['memory'][0]
    mem_pad = jnp.pad(mem, ((0, 128 - mem.shape[0]), (0, 0)))
    headw = jnp.concatenate(
        [p['mood_w'], p['pers_w'],
         jnp.zeros((128 - p['mood_w'].shape[0] - p['pers_w'].shape[0], d3),
                   _F32)], axis=0)
    headb = jnp.concatenate(
        [p['mood_b'], p['pers_b'],
         jnp.zeros((128 - p['mood_b'].shape[0] - p['pers_b'].shape[0],),
                   _F32)], axis=0)

    tb = 32
    n1 = t // tb
    xg0 = pl.pallas_call(
        _make_prep_body(mem.shape[0]),
        grid=(n1,),
        in_specs=[pl.BlockSpec((bsz, tb, d2), lambda j: (0, j, 0))]
        + [_vspec()] * 12,
        out_specs=pl.BlockSpec((tb, bsz, h4), lambda j: (j, 0, 0)),
        out_shape=jax.ShapeDtypeStruct((t, bsz, h4), _F32),
        scratch_shapes=[pltpu.VMEM((128, dm), _BF16),
                        pltpu.VMEM((128, dm), _BF16)],
        compiler_params=pltpu.CompilerParams(
            dimension_semantics=("arbitrary",),
            vmem_limit_bytes=_VMEM_LIMIT),
        name="l3ap_prep",
    )(z2, mem_pad, bf(p['mk_w'].T), row(p['mk_b']), bf(p['mv_w'].T),
      row(p['mv_b']), bf(p['in_w'].T), row(p['in_b']), bf(p['mq_w'].T),
      row(p['mq_b']), bf(p['wih0'][:, :d3]), bf(p['wih0'][:, d3:]),
      row(p['bih0'] + p['bhh0']))

    ts = 16
    h1seq = pl.pallas_call(
        _scan_body,
        grid=(t // ts,),
        in_specs=[pl.BlockSpec((ts, bsz, h4), lambda s: (s, 0, 0)),
                  _vspec(), _vspec(), _vspec(), _vspec()],
        out_specs=pl.BlockSpec((ts, bsz, h), lambda s: (s, 0, 0)),
        out_shape=jax.ShapeDtypeStruct((t, bsz, h), _F32),
        scratch_shapes=[pltpu.VMEM((bsz, h), _BF16),
                        pltpu.VMEM((bsz, h), _F32),
                        pltpu.VMEM((bsz, h), _BF16),
                        pltpu.VMEM((bsz, h), _F32),
                        pltpu.VMEM((ts * bsz, h), _BF16),
                        pltpu.VMEM((ts * bsz, h4), _F32)],
        compiler_params=pltpu.CompilerParams(
            dimension_semantics=("arbitrary",),
            vmem_limit_bytes=_VMEM_LIMIT),
        name="l3ap_scan",
    )(xg0, p['whh0'].astype(_BF16).T, bf(p['wih1']),
      p['whh1'].astype(_BF16).T, row(p['bih1'] + p['bhh1']))

    rows = t * bsz
    rb = 256
    n3 = rows // rb
    nt = rb // bsz
    hflat = h1seq.reshape(rows, h)
    enc, heads, pred = pl.pallas_call(
        _make_enc_body(n3),
        grid=(n3,),
        in_specs=[pl.BlockSpec((rb, h), lambda j: (j, 0))]
        + [_vspec()] * 16,
        out_specs=[pl.BlockSpec((bsz, nt, d3), lambda j: (0, j, 0)),
                   pl.BlockSpec((bsz, nt, 128), lambda j: (0, j, 0)),
                   pl.BlockSpec((bsz, d3), lambda j: (0, 0))],
        out_shape=[jax.ShapeDtypeStruct((bsz, t, d3), _F32),
                   jax.ShapeDtypeStruct((bsz, t, 128), _F32),
                   jax.ShapeDtypeStruct((bsz, d3), _F32)],
        compiler_params=pltpu.CompilerParams(
            dimension_semantics=("arbitrary",),
            vmem_limit_bytes=_VMEM_LIMIT),
        name="l3ap_enc",
    )(hflat, bf(p['e1_w']), row(p['e1_b']), row(p['ln1_g']),
      row(p['ln1_b']), bf(p['e2_w']), row(p['e2_b']), row(p['ln2_g']),
      row(p['ln2_b']), bf(p['e3_w']), row(p['e3_b']), bf(headw),
      row(headb), bf(p['p1_w']), row(p['p1_b']), bf(p['p2_w']),
      row(p['p2_b']))

    mood = heads[..., :p['mood_w'].shape[0]]
    pers = heads[..., p['mood_w'].shape[0]:
                 p['mood_w'].shape[0] + p['pers_w'].shape[0]]
    return (enc, mood, pers, pred)
